# Initial kernel scaffold; baseline (speedup 1.0000x reference)
import jax
import jax.numpy as jnp
from jax.experimental import pallas as pl


def kernel(x):
    raise NotImplementedError("write your pallas kernel here")


# speedup vs baseline: 1.0230x; 1.0230x over previous
import functools

import numpy as np
import jax
import jax.numpy as jnp
from jax import lax
from jax.experimental import pallas as pl
from jax.experimental.pallas import tpu as pltpu
from jax.experimental.pallas import tpu_sc as plsc

_B, _S, _F = 4096, 50, 64
_SP = 64
_RATE = 0.6

_NW = 32
_RW = _B * _F // _NW
_CH = 128
_CPW = _RW // _CH
_SLAB = 8
_SLAB_ROWS = _SLAB * _CH
_NSLABS = _CPW // _SLAB

_BB = 128
_NBLK = _B // _BB


def _make_consts():
    with jax.default_device(jax.local_devices(backend="cpu")[0]):
        return _make_consts_impl()


def _make_consts_impl():
    rk = jax.random.key(42)
    k1, k2, k3, k4 = jax.random.split(rk, 4)
    m1 = np.asarray(jax.random.uniform(k1, (_B, _S, _F)) < _RATE)
    p1 = np.asarray(
        jax.vmap(lambda k: jax.random.permutation(k, _B))(jax.random.split(k2, _F)))
    m2 = np.asarray(jax.random.uniform(k3, (_B, _S, _F)) < _RATE)
    p2 = np.asarray(
        jax.vmap(lambda k: jax.random.permutation(k, _B))(jax.random.split(k4, _F)))

    def row_idx(p):
        g = p.T.astype(np.int32) * _F + np.arange(_F, dtype=np.int32)[None, :]
        return np.ascontiguousarray(g.reshape(_NW, _CPW, _CH))

    return m1, m2, row_idx(p1), row_idx(p2)


_M1, _M2, _I1, _I2 = _make_consts()


def _tc_transpose(x):
    def body(x_ref, o_ref):
        t = jnp.transpose(x_ref[...], (0, 2, 1)).reshape(_BB * _F, _S)
        o_ref[...] = jnp.pad(t, ((0, 0), (0, _SP - _S)))

    return pl.pallas_call(
        body,
        grid=(_NBLK,),
        in_specs=[pl.BlockSpec((_BB, _S, _F), lambda i: (i, 0, 0))],
        out_specs=pl.BlockSpec((_BB * _F, _SP), lambda i: (i, 0)),
        out_shape=jax.ShapeDtypeStruct((_B * _F, _SP), jnp.float32),
    )(x)


def _sc_gather(xt, idx1, idx2):
    mesh = plsc.VectorSubcoreMesh(core_axis_name="c", subcore_axis_name="s")

    @functools.partial(
        pl.kernel,
        out_type=(jax.ShapeDtypeStruct((_B * _F, _SP), jnp.float32),) * 2,
        mesh=mesh,
        scratch_types=[
            pltpu.VMEM((_CPW, _CH), jnp.int32),
            pltpu.VMEM((_CPW, _CH), jnp.int32),
            pltpu.VMEM((_SLAB_ROWS, _SP), jnp.float32),
            pltpu.SemaphoreType.DMA,
        ],
        compiler_params=pltpu.CompilerParams(use_tc_tiling_on_sc=False),
    )
    def k(xt_hbm, i1_hbm, i2_hbm, o1_hbm, o2_hbm, iv1, iv2, buf, sem):
        wid = lax.axis_index("s") * 2 + lax.axis_index("c")
        pltpu.sync_copy(i1_hbm.at[wid], iv1)
        pltpu.sync_copy(i2_hbm.at[wid], iv2)
        base = wid * _RW
        for iv, out in ((iv1, o1_hbm), (iv2, o2_hbm)):
            @pl.loop(0, _NSLABS)
            def _slab(s, iv=iv, out=out):
                cps = [
                    pltpu.async_copy(
                        xt_hbm.at[iv.at[s * _SLAB + j]],
                        buf.at[pl.ds(j * _CH, _CH)],
                        sem,
                    )
                    for j in range(_SLAB)
                ]
                for cp in cps:
                    cp.wait()
                pltpu.sync_copy(
                    buf, out.at[pl.ds(base + s * _SLAB_ROWS, _SLAB_ROWS)])

    return k(xt, idx1, idx2)


def _tc_select(shuf1, shuf2, x, m1, m2):
    def body(s1_ref, s2_ref, x_ref, m1_ref, m2_ref, o1_ref, o2_ref, om_ref):
        xb = x_ref[...]
        s1 = jnp.transpose(s1_ref[...].reshape(_BB, _F, _SP)[:, :, :_S], (0, 2, 1))
        s2 = jnp.transpose(s2_ref[...].reshape(_BB, _F, _SP)[:, :, :_S], (0, 2, 1))
        mb1 = m1_ref[...]
        o1_ref[...] = jnp.where(mb1, s1, xb)
        o2_ref[...] = jnp.where(m2_ref[...], s2, xb)
        om_ref[...] = mb1

    spec3 = pl.BlockSpec((_BB, _S, _F), lambda i: (i, 0, 0))
    spec2 = pl.BlockSpec((_BB * _F, _SP), lambda i: (i, 0))
    f3 = jax.ShapeDtypeStruct((_B, _S, _F), jnp.float32)
    b3 = jax.ShapeDtypeStruct((_B, _S, _F), jnp.bool_)
    return pl.pallas_call(
        body,
        grid=(_NBLK,),
        in_specs=[spec2, spec2, spec3, spec3, spec3],
        out_specs=[spec3, spec3, spec3],
        out_shape=[f3, f3, b3],
    )(shuf1, shuf2, x, m1, m2)


def kernel(x):
    xt = _tc_transpose(x)
    s1, s2 = _sc_gather(xt, _I1, _I2)
    corrupted, positive, mask = _tc_select(s1, s2, x, _M1, _M2)
    return corrupted, positive, mask, x


# speedup vs baseline: 1.1571x; 1.1310x over previous
import functools

import numpy as np
import jax
import jax.numpy as jnp
from jax import lax
from jax.experimental import pallas as pl
from jax.experimental.pallas import tpu as pltpu
from jax.experimental.pallas import tpu_sc as plsc

_B, _S, _F = 4096, 50, 64
_SP = 64
_RATE = 0.6

_NW = 32
_RW = _B * _F // _NW
_CH = 128
_CPW = _RW // _CH
_SLAB = 8
_SLAB_ROWS = _SLAB * _CH
_NSLABS = _CPW // _SLAB

_BB = 128
_NBLK = _B // _BB


def _make_consts():
    with jax.default_device(jax.local_devices(backend="cpu")[0]):
        return _make_consts_impl()


def _make_consts_impl():
    rk = jax.random.key(42)
    k1, k2, k3, k4 = jax.random.split(rk, 4)
    m1 = np.asarray(jax.random.uniform(k1, (_B, _S, _F)) < _RATE)
    p1 = np.asarray(
        jax.vmap(lambda k: jax.random.permutation(k, _B))(jax.random.split(k2, _F)))
    m2 = np.asarray(jax.random.uniform(k3, (_B, _S, _F)) < _RATE)
    p2 = np.asarray(
        jax.vmap(lambda k: jax.random.permutation(k, _B))(jax.random.split(k4, _F)))

    def row_idx(p):
        g = p.T.astype(np.int32) * _F + np.arange(_F, dtype=np.int32)[None, :]
        return np.ascontiguousarray(g.reshape(_NW, _CPW, _CH))

    return (m1, m2, m1.astype(np.uint8), m2.astype(np.uint8),
            row_idx(p1), row_idx(p2))


_M1, _M2, _M1U, _M2U, _I1, _I2 = _make_consts()


def _tc_transpose(x):
    def body(x_ref, o_ref):
        t = jnp.transpose(x_ref[...], (0, 2, 1)).reshape(_BB * _F, _S)
        o_ref[...] = jnp.pad(t, ((0, 0), (0, _SP - _S)))

    return pl.pallas_call(
        body,
        grid=(_NBLK,),
        in_specs=[pl.BlockSpec((_BB, _S, _F), lambda i: (i, 0, 0))],
        out_specs=pl.BlockSpec((_BB * _F, _SP), lambda i: (i, 0)),
        out_shape=jax.ShapeDtypeStruct((_B * _F, _SP), jnp.float32),
    )(x)


def _sc_gather(xt, idx1, idx2):
    mesh = plsc.VectorSubcoreMesh(core_axis_name="c", subcore_axis_name="s")

    @functools.partial(
        pl.kernel,
        out_type=(jax.ShapeDtypeStruct((_B * _F, _SP), jnp.float32),) * 2,
        mesh=mesh,
        scratch_types=[
            pltpu.VMEM((_CPW, _CH), jnp.int32),
            pltpu.VMEM((_CPW, _CH), jnp.int32),
            pltpu.VMEM((_SLAB_ROWS, _SP), jnp.float32),
            pltpu.SemaphoreType.DMA,
        ],
        compiler_params=pltpu.CompilerParams(use_tc_tiling_on_sc=False),
    )
    def k(xt_hbm, i1_hbm, i2_hbm, o1_hbm, o2_hbm, iv1, iv2, buf, sem):
        wid = lax.axis_index("s") * 2 + lax.axis_index("c")
        pltpu.sync_copy(i1_hbm.at[wid], iv1)
        pltpu.sync_copy(i2_hbm.at[wid], iv2)
        base = wid * _RW
        for iv, out in ((iv1, o1_hbm), (iv2, o2_hbm)):
            @pl.loop(0, _NSLABS)
            def _slab(s, iv=iv, out=out):
                cps = [
                    pltpu.async_copy(
                        xt_hbm.at[iv.at[s * _SLAB + j]],
                        buf.at[pl.ds(j * _CH, _CH)],
                        sem,
                    )
                    for j in range(_SLAB)
                ]
                for cp in cps:
                    cp.wait()
                pltpu.sync_copy(
                    buf, out.at[pl.ds(base + s * _SLAB_ROWS, _SLAB_ROWS)])

    return k(xt, idx1, idx2)


def _tc_select(shuf1, shuf2, x, m1, m2):
    def body(s1_ref, s2_ref, x_ref, m1_ref, m2_ref, o1_ref, o2_ref):
        xb = x_ref[...]
        s1 = jnp.transpose(s1_ref[...].reshape(_BB, _F, _SP)[:, :, :_S], (0, 2, 1))
        s2 = jnp.transpose(s2_ref[...].reshape(_BB, _F, _SP)[:, :, :_S], (0, 2, 1))
        o1_ref[...] = jnp.where(m1_ref[...] != 0, s1, xb)
        o2_ref[...] = jnp.where(m2_ref[...] != 0, s2, xb)

    spec3 = pl.BlockSpec((_BB, _S, _F), lambda i: (i, 0, 0))
    spec2 = pl.BlockSpec((_BB * _F, _SP), lambda i: (i, 0))
    f3 = jax.ShapeDtypeStruct((_B, _S, _F), jnp.float32)
    return pl.pallas_call(
        body,
        grid=(_NBLK,),
        in_specs=[spec2, spec2, spec3, spec3, spec3],
        out_specs=[spec3, spec3],
        out_shape=[f3, f3],
    )(shuf1, shuf2, x, m1, m2)


def kernel(x):
    xt = _tc_transpose(x)
    s1, s2 = _sc_gather(xt, _I1, _I2)
    corrupted, positive = _tc_select(s1, s2, x, _M1U, _M2U)
    return corrupted, positive, jnp.asarray(_M1), x


# speedup vs baseline: 1.6398x; 1.4172x over previous
import functools

import numpy as np
import jax
import jax.numpy as jnp
from jax import lax
from jax.experimental import pallas as pl
from jax.experimental.pallas import tpu as pltpu
from jax.experimental.pallas import tpu_sc as plsc

_B, _S, _F = 4096, 50, 64
_SP = 64
_HF = _F // 2
_RATE = 0.6

_NW = 32
_RW = _B * _F // _NW
_CH = 128
_CPW = _RW // _CH
_SLAB = 8
_SLAB_ROWS = _SLAB * _CH
_NSLABS = _CPW // _SLAB

_BB = 128
_NBLK = _B // _BB


def _make_consts():
    with jax.default_device(jax.local_devices(backend="cpu")[0]):
        return _make_consts_impl()


def _row_of(b, f):
    return (b * _HF + f % _HF) * 2 + f // _HF


def _make_consts_impl():
    rk = jax.random.key(42)
    k1, k2, k3, k4 = jax.random.split(rk, 4)
    m1 = np.asarray(jax.random.uniform(k1, (_B, _S, _F)) < _RATE)
    p1 = np.asarray(
        jax.vmap(lambda k: jax.random.permutation(k, _B))(jax.random.split(k2, _F)))
    m2 = np.asarray(jax.random.uniform(k3, (_B, _S, _F)) < _RATE)
    p2 = np.asarray(
        jax.vmap(lambda k: jax.random.permutation(k, _B))(jax.random.split(k4, _F)))

    def row_idx(p):
        k = np.arange(_B * _F, dtype=np.int64)
        q, h = k // 2, k % 2
        b, fm = q // _HF, q % _HF
        f = h * _HF + fm
        src = _row_of(p[f, b].astype(np.int64), f)
        return np.ascontiguousarray(
            src.astype(np.int32).reshape(_NW, _CPW, _CH))

    return (m1, m2, m1.astype(np.uint8), m2.astype(np.uint8),
            row_idx(p1), row_idx(p2))


_M1, _M2, _M1U, _M2U, _I1, _I2 = _make_consts()


def _tc_transpose(x):
    def body(x_ref, o_ref):
        t = jnp.transpose(x_ref[...], (0, 2, 1))
        t = jnp.pad(t, ((0, 0), (0, 0), (0, _SP - _S)))
        lo = t[:, :_HF, :].reshape(_BB * _HF, _SP)
        hi = t[:, _HF:, :].reshape(_BB * _HF, _SP)
        o_ref[...] = jnp.concatenate([lo, hi], axis=1)

    return pl.pallas_call(
        body,
        grid=(_NBLK,),
        in_specs=[pl.BlockSpec((_BB, _S, _F), lambda i: (i, 0, 0))],
        out_specs=pl.BlockSpec((_BB * _HF, 2 * _SP), lambda i: (i, 0)),
        out_shape=jax.ShapeDtypeStruct((_B * _HF, 2 * _SP), jnp.float32),
    )(x)


def _sc_gather(xt, idx1, idx2):
    mesh = plsc.VectorSubcoreMesh(core_axis_name="c", subcore_axis_name="s")

    @functools.partial(
        pl.kernel,
        out_type=(jax.ShapeDtypeStruct((_B * _F, _SP), jnp.float32),) * 2,
        mesh=mesh,
        scratch_types=[
            pltpu.VMEM((_CPW, _CH), jnp.int32),
            pltpu.VMEM((_CPW, _CH), jnp.int32),
            pltpu.VMEM((_SLAB_ROWS, _SP), jnp.float32),
            pltpu.SemaphoreType.DMA,
        ],
        compiler_params=pltpu.CompilerParams(use_tc_tiling_on_sc=False),
    )
    def k(xt_hbm, i1_hbm, i2_hbm, o1_hbm, o2_hbm, iv1, iv2, buf, sem):
        wid = lax.axis_index("s") * 2 + lax.axis_index("c")
        pltpu.sync_copy(i1_hbm.at[wid], iv1)
        pltpu.sync_copy(i2_hbm.at[wid], iv2)
        base = wid * _RW
        for iv, out in ((iv1, o1_hbm), (iv2, o2_hbm)):
            @pl.loop(0, _NSLABS)
            def _slab(s, iv=iv, out=out):
                cps = [
                    pltpu.async_copy(
                        xt_hbm.at[iv.at[s * _SLAB + j]],
                        buf.at[pl.ds(j * _CH, _CH)],
                        sem,
                    )
                    for j in range(_SLAB)
                ]
                for cp in cps:
                    cp.wait()
                pltpu.sync_copy(
                    buf, out.at[pl.ds(base + s * _SLAB_ROWS, _SLAB_ROWS)])

    return k(xt, idx1, idx2)


def _tc_select(shuf1, shuf2, x, m1, m2):
    def unpack(ref):
        lo = ref[:, :_SP].reshape(_BB, _HF, _SP)[:, :, :_S]
        hi = ref[:, _SP:].reshape(_BB, _HF, _SP)[:, :, :_S]
        return jnp.concatenate(
            [jnp.transpose(lo, (0, 2, 1)), jnp.transpose(hi, (0, 2, 1))],
            axis=2)

    def body(s1_ref, s2_ref, x_ref, m1_ref, m2_ref, o1_ref, o2_ref):
        xb = x_ref[...]
        o1_ref[...] = jnp.where(m1_ref[...] != 0, unpack(s1_ref), xb)
        o2_ref[...] = jnp.where(m2_ref[...] != 0, unpack(s2_ref), xb)

    spec3 = pl.BlockSpec((_BB, _S, _F), lambda i: (i, 0, 0))
    spec2 = pl.BlockSpec((_BB * _HF, 2 * _SP), lambda i: (i, 0))
    f3 = jax.ShapeDtypeStruct((_B, _S, _F), jnp.float32)
    return pl.pallas_call(
        body,
        grid=(_NBLK,),
        in_specs=[spec2, spec2, spec3, spec3, spec3],
        out_specs=[spec3, spec3],
        out_shape=[f3, f3],
    )(shuf1, shuf2, x, m1, m2)


def kernel(x):
    tbl = _tc_transpose(x)
    s1, s2 = _sc_gather(tbl.reshape(_B * _F, _SP), _I1, _I2)
    corrupted, positive = _tc_select(
        s1.reshape(_B * _HF, 2 * _SP), s2.reshape(_B * _HF, 2 * _SP),
        x, _M1U, _M2U)
    return corrupted, positive, jnp.asarray(_M1), x


# speedup vs baseline: 2.5070x; 1.5288x over previous
import functools

import numpy as np
import jax
import jax.numpy as jnp
from jax import lax
from jax.experimental import pallas as pl
from jax.experimental.pallas import tpu as pltpu
from jax.experimental.pallas import tpu_sc as plsc

_B, _S, _F = 4096, 50, 64
_SP = 64
_RATE = 0.6

_NW = 32
_RW = _B * _F // _NW
_CH = 128
_CPW = _RW // _CH
_SLAB = 8
_SLAB_ROWS = _SLAB * _CH
_NSLABS = _CPW // _SLAB

_FB = 8
_FBP = _FB // 2
_NBLK = _F // _FB
_BBLK = 2048


def _make_consts():
    with jax.default_device(jax.local_devices(backend="cpu")[0]):
        return _make_consts_impl()


def _row_of(b, f):
    return (f // 2) * (2 * _B) + b * 2 + f % 2


def _make_consts_impl():
    rk = jax.random.key(42)
    k1, k2, k3, k4 = jax.random.split(rk, 4)
    m1 = np.asarray(jax.random.uniform(k1, (_B, _S, _F)) < _RATE)
    p1 = np.asarray(
        jax.vmap(lambda k: jax.random.permutation(k, _B))(jax.random.split(k2, _F)))
    m2 = np.asarray(jax.random.uniform(k3, (_B, _S, _F)) < _RATE)
    p2 = np.asarray(
        jax.vmap(lambda k: jax.random.permutation(k, _B))(jax.random.split(k4, _F)))

    def row_idx(p):
        k = np.arange(_B * _F, dtype=np.int64)
        f2, r = k // (2 * _B), k % (2 * _B)
        b, h = r // 2, r % 2
        f = 2 * f2 + h
        src = _row_of(p[f, b].astype(np.int64), f)
        return np.ascontiguousarray(
            src.astype(np.int32).reshape(_NW, _CPW, _CH))

    def mt(m):
        t = np.transpose(m, (1, 2, 0)).astype(np.uint8)
        return np.ascontiguousarray(t.reshape(_S, _F // _FB, _FB, _B))

    return m1, mt(m1), mt(m2), row_idx(p1), row_idx(p2)


_M1, _M1T, _M2T, _I1, _I2 = _make_consts()


def _tc_pack(xt):
    def body(x_ref, o_ref):
        for fi in range(_FB):
            t = jnp.transpose(x_ref[:, fi, :], (1, 0))
            o_ref[fi // 2, :, (fi % 2) * _SP:(fi % 2) * _SP + _S] = t

    return pl.pallas_call(
        body,
        grid=(_NBLK, _B // _BBLK),
        in_specs=[pl.BlockSpec((_S, _FB, _BBLK), lambda i, j: (0, i, j))],
        out_specs=pl.BlockSpec((_FBP, _BBLK, 2 * _SP), lambda i, j: (i, j, 0)),
        out_shape=jax.ShapeDtypeStruct((_F // 2, _B, 2 * _SP), jnp.float32),
    )(xt)


def _sc_gather(xt, idx1, idx2):
    mesh = plsc.VectorSubcoreMesh(core_axis_name="c", subcore_axis_name="s")

    @functools.partial(
        pl.kernel,
        out_type=(jax.ShapeDtypeStruct((_B * _F, _SP), jnp.float32),) * 2,
        mesh=mesh,
        scratch_types=[
            pltpu.VMEM((_CPW, _CH), jnp.int32),
            pltpu.VMEM((_CPW, _CH), jnp.int32),
            pltpu.VMEM((_SLAB_ROWS, _SP), jnp.float32),
            pltpu.SemaphoreType.DMA,
        ],
        compiler_params=pltpu.CompilerParams(use_tc_tiling_on_sc=False),
    )
    def k(xt_hbm, i1_hbm, i2_hbm, o1_hbm, o2_hbm, iv1, iv2, buf, sem):
        wid = lax.axis_index("s") * 2 + lax.axis_index("c")
        pltpu.sync_copy(i1_hbm.at[wid], iv1)
        pltpu.sync_copy(i2_hbm.at[wid], iv2)
        base = wid * _RW
        for iv, out in ((iv1, o1_hbm), (iv2, o2_hbm)):
            @pl.loop(0, _NSLABS)
            def _slab(s, iv=iv, out=out):
                cps = [
                    pltpu.async_copy(
                        xt_hbm.at[iv.at[s * _SLAB + j]],
                        buf.at[pl.ds(j * _CH, _CH)],
                        sem,
                    )
                    for j in range(_SLAB)
                ]
                for cp in cps:
                    cp.wait()
                pltpu.sync_copy(
                    buf, out.at[pl.ds(base + s * _SLAB_ROWS, _SLAB_ROWS)])

    return k(xt, idx1, idx2)


def _tc_select(shuf1, shuf2, xt, m1t, m2t):
    def unpack(ref):
        lo = ref[:, :, :_S]
        hi = ref[:, :, _SP:_SP + _S]
        lo = jnp.transpose(lo, (2, 0, 1))
        hi = jnp.transpose(hi, (2, 0, 1))
        t = jnp.stack([lo, hi], axis=2)
        return t.reshape(_S, _FB, _BBLK)

    def body(s1_ref, s2_ref, x_ref, m1_ref, m2_ref, o1_ref, o2_ref):
        xb = x_ref[...]
        m1 = m1_ref[:, 0]
        m2 = m2_ref[:, 0]
        o1_ref[...] = jnp.where(m1 != 0, unpack(s1_ref), xb)
        o2_ref[...] = jnp.where(m2 != 0, unpack(s2_ref), xb)

    spec3 = pl.BlockSpec((_S, _FB, _BBLK), lambda i, j: (0, i, j))
    spec3m = pl.BlockSpec((_S, 1, _FB, _BBLK), lambda i, j: (0, i, 0, j))
    spec2 = pl.BlockSpec((_FBP, _BBLK, 2 * _SP), lambda i, j: (i, j, 0))
    f3 = jax.ShapeDtypeStruct((_S, _F, _B), jnp.float32)
    return pl.pallas_call(
        body,
        grid=(_NBLK, _B // _BBLK),
        in_specs=[spec2, spec2, spec3, spec3m, spec3m],
        out_specs=[spec3, spec3],
        out_shape=[f3, f3],
    )(shuf1, shuf2, xt, m1t, m2t)


def kernel(x):
    xt = jnp.transpose(x, (1, 2, 0))
    tbl = _tc_pack(xt)
    s1, s2 = _sc_gather(tbl.reshape(_B * _F, _SP), _I1, _I2)
    o1t, o2t = _tc_select(
        s1.reshape(_F // 2, _B, 2 * _SP), s2.reshape(_F // 2, _B, 2 * _SP),
        xt, _M1T, _M2T)
    corrupted = jnp.transpose(o1t, (2, 0, 1))
    positive = jnp.transpose(o2t, (2, 0, 1))
    return corrupted, positive, jnp.asarray(_M1), x


# speedup vs baseline: 2.6721x; 1.0659x over previous
import functools

import numpy as np
import jax
import jax.numpy as jnp
from jax import lax
from jax.experimental import pallas as pl
from jax.experimental.pallas import tpu as pltpu
from jax.experimental.pallas import tpu_sc as plsc

_B, _S, _F = 4096, 50, 64
_SP = 64
_RATE = 0.6

_NW = 32
_RW = _B * _F // _NW
_CH = 128
_CPW = _RW // _CH
_SLAB = 8
_SLAB_ROWS = _SLAB * _CH
_NSLABS = _CPW // _SLAB

_FB = 8
_FBP = _FB // 2
_NBLK = _F // _FB
_BBLK = 2048


def _make_consts():
    with jax.default_device(jax.local_devices(backend="cpu")[0]):
        return _make_consts_impl()


def _row_of(b, f):
    return (f // 2) * (2 * _B) + b * 2 + f % 2


def _make_consts_impl():
    rk = jax.random.key(42)
    k1, k2, k3, k4 = jax.random.split(rk, 4)
    m1 = np.asarray(jax.random.uniform(k1, (_B, _S, _F)) < _RATE)
    p1 = np.asarray(
        jax.vmap(lambda k: jax.random.permutation(k, _B))(jax.random.split(k2, _F)))
    m2 = np.asarray(jax.random.uniform(k3, (_B, _S, _F)) < _RATE)
    p2 = np.asarray(
        jax.vmap(lambda k: jax.random.permutation(k, _B))(jax.random.split(k4, _F)))

    def row_idx(p):
        k = np.arange(_B * _F, dtype=np.int64)
        f2, r = k // (2 * _B), k % (2 * _B)
        b, h = r // 2, r % 2
        f = 2 * f2 + h
        src = _row_of(p[f, b].astype(np.int64), f)
        return np.ascontiguousarray(
            src.astype(np.int32).reshape(_NW, _CPW, _CH))

    def mt(m):
        t = np.transpose(m, (1, 2, 0)).astype(np.uint8)
        return np.ascontiguousarray(t.reshape(_S, _F // _FB, _FB, _B))

    return m1, mt(m1), mt(m2), row_idx(p1), row_idx(p2)


_M1, _M1T, _M2T, _I1, _I2 = _make_consts()


def _tc_pack(xt):
    def body(x_ref, o_ref):
        for fi in range(_FB):
            t = jnp.transpose(x_ref[:, fi, :], (1, 0))
            o_ref[fi // 2, :, (fi % 2) * _SP:(fi % 2) * _SP + _S] = t

    return pl.pallas_call(
        body,
        grid=(_NBLK, _B // _BBLK),
        in_specs=[pl.BlockSpec((_S, _FB, _BBLK), lambda i, j: (0, i, j))],
        out_specs=pl.BlockSpec((_FBP, _BBLK, 2 * _SP), lambda i, j: (i, j, 0)),
        out_shape=jax.ShapeDtypeStruct((_F // 2, _B, 2 * _SP), jnp.float32),
    )(xt)


def _sc_gather(xt, idx):
    mesh = plsc.VectorSubcoreMesh(core_axis_name="c", subcore_axis_name="s")

    @functools.partial(
        pl.kernel,
        out_type=jax.ShapeDtypeStruct((_B * _F, _SP), jnp.float32),
        mesh=mesh,
        scratch_types=[
            pltpu.VMEM((_CPW, _CH), jnp.int32),
            pltpu.VMEM((_SLAB_ROWS, _SP), jnp.float32),
            pltpu.SemaphoreType.DMA,
        ],
        compiler_params=pltpu.CompilerParams(use_tc_tiling_on_sc=False),
    )
    def k(xt_hbm, i_hbm, o_hbm, iv, buf, sem):
        wid = lax.axis_index("s") * 2 + lax.axis_index("c")
        pltpu.sync_copy(i_hbm.at[wid], iv)
        base = wid * _RW

        @pl.loop(0, _NSLABS)
        def _slab(s):
            cps = [
                pltpu.async_copy(
                    xt_hbm.at[iv.at[s * _SLAB + j]],
                    buf.at[pl.ds(j * _CH, _CH)],
                    sem,
                )
                for j in range(_SLAB)
            ]
            for cp in cps:
                cp.wait()
            pltpu.sync_copy(
                buf, o_hbm.at[pl.ds(base + s * _SLAB_ROWS, _SLAB_ROWS)])

    return k(xt, idx)


def _tc_select(shuf, xt, mt):
    def unpack(ref):
        lo = ref[:, :, :_S]
        hi = ref[:, :, _SP:_SP + _S]
        lo = jnp.transpose(lo, (2, 0, 1))
        hi = jnp.transpose(hi, (2, 0, 1))
        t = jnp.stack([lo, hi], axis=2)
        return t.reshape(_S, _FB, _BBLK)

    def body(s_ref, x_ref, m_ref, o_ref):
        o_ref[...] = jnp.where(m_ref[:, 0] != 0, unpack(s_ref), x_ref[...])

    spec3 = pl.BlockSpec((_S, _FB, _BBLK), lambda i, j: (0, i, j))
    spec3m = pl.BlockSpec((_S, 1, _FB, _BBLK), lambda i, j: (0, i, 0, j))
    spec2 = pl.BlockSpec((_FBP, _BBLK, 2 * _SP), lambda i, j: (i, j, 0))
    f3 = jax.ShapeDtypeStruct((_S, _F, _B), jnp.float32)
    return pl.pallas_call(
        body,
        grid=(_NBLK, _B // _BBLK),
        in_specs=[spec2, spec3, spec3m],
        out_specs=spec3,
        out_shape=f3,
    )(shuf, xt, mt)


def kernel(x):
    xt = jnp.transpose(x, (1, 2, 0))
    tbl = _tc_pack(xt)
    tbl2 = tbl.reshape(_B * _F, _SP)
    s1 = _sc_gather(tbl2, _I1)
    s2 = _sc_gather(tbl2, _I2)
    o1t = _tc_select(s1.reshape(_F // 2, _B, 2 * _SP), xt, _M1T)
    o2t = _tc_select(s2.reshape(_F // 2, _B, 2 * _SP), xt, _M2T)
    corrupted = jnp.transpose(o1t, (2, 0, 1))
    positive = jnp.transpose(o2t, (2, 0, 1))
    return corrupted, positive, jnp.asarray(_M1), x
